# baseline (speedup 1.0000x reference)
import functools

import jax
import jax.numpy as jnp
from jax.experimental import pallas as pl
from jax.experimental.pallas import tpu as pltpu
from jax.experimental.pallas import tpu_sc as plsc

_N = 160000
_D = 128
_M = 1024
_B = 2000
_NB = _N // _B

_NW = 32
_GRP = 40
_BATCH = 128
_CHUNK = _GRP * _BATCH
_NPAD = _NW * _CHUNK
_f32 = jnp.float32


def _tc_body(dep_ref, b1_ref, feats_ref, w0_ref, w1_ref,
             atomen_ref, ahier_ref, vals_ref):
    f0 = feats_ref[0]
    f1 = feats_ref[1]
    w0 = w0_ref[...]
    w1 = w1_ref[...]
    dep = dep_ref[0, 0]
    b1 = b1_ref[0, 0]
    f0b = f0.astype(jnp.bfloat16).astype(_f32)
    f1b = f1.astype(jnp.bfloat16).astype(_f32)
    w0b = w0.astype(jnp.bfloat16).astype(_f32)
    w1b = w1.astype(jnp.bfloat16).astype(_f32)
    pe0 = jnp.sum(f0b * w0b, axis=1, keepdims=True) + dep
    pe1 = jnp.sum(f1b * w1b, axis=1, keepdims=True) + b1
    e0s = pe0 * pe0
    e1s = pe1 * pe1
    den = e0s + e1s
    hier = e1s / den
    atomen_ref[...] = pe0 + pe1
    ahier_ref[...] = hier
    vals_ref[...] = jnp.concatenate(
        [pe0, pe1, hier, e1s, den, jnp.zeros((_B, 11), _f32)], axis=1)


def _run_tc(dep, b1r, feats, W0, W1):
    return pl.pallas_call(
        _tc_body,
        grid=(_NB,),
        in_specs=[
            pl.BlockSpec(memory_space=pltpu.SMEM),
            pl.BlockSpec(memory_space=pltpu.SMEM),
            pl.BlockSpec((2, _B, _D), lambda i: (0, i, 0)),
            pl.BlockSpec((1, _D), lambda i: (0, 0)),
            pl.BlockSpec((1, _D), lambda i: (0, 0)),
        ],
        out_specs=[
            pl.BlockSpec((_B, 1), lambda i: (i, 0)),
            pl.BlockSpec((_B, 1), lambda i: (i, 0)),
            pl.BlockSpec((_B, 16), lambda i: (i, 0)),
        ],
        out_shape=[
            jax.ShapeDtypeStruct((_N, 1), _f32),
            jax.ShapeDtypeStruct((_N, 1), _f32),
            jax.ShapeDtypeStruct((_NPAD, 16), _f32),
        ],
    )(dep, b1r, feats, W0, W1)


def _sc_body(vals_hbm, mol_hbm, zeros_hbm, out_hbm, idx_v, vals_v, acc_sh):
    c = jax.lax.axis_index("c")
    s = jax.lax.axis_index("s")
    wid = s * 2 + c
    pltpu.sync_copy(mol_hbm.at[wid], idx_v)
    pltpu.sync_copy(vals_hbm.at[pl.ds(wid * _CHUNK, _CHUNK)], vals_v)

    @pl.when(s == 0)
    def _zero():
        pltpu.sync_copy(zeros_hbm, acc_sh)

    plsc.subcore_barrier()

    def _grp(g, carry):
        pltpu.sync_copy(vals_v.at[pl.ds(g * _BATCH, _BATCH)],
                        acc_sh.at[idx_v.at[g]], add=True)
        return carry

    jax.lax.fori_loop(0, _GRP, _grp, 0)
    plsc.subcore_barrier()

    @pl.when(s == 0)
    def _flush():
        pltpu.sync_copy(acc_sh.at[pl.ds(0, _M)], out_hbm.at[c])


_sc_segsum = functools.partial(
    pl.kernel,
    out_type=jax.ShapeDtypeStruct((2, _M, 16), _f32),
    mesh=plsc.VectorSubcoreMesh(core_axis_name="c", subcore_axis_name="s"),
    compiler_params=pltpu.CompilerParams(use_tc_tiling_on_sc=False),
    scratch_types=[
        pltpu.VMEM((_GRP, _BATCH), jnp.int32),
        pltpu.VMEM((_CHUNK, 16), _f32),
        pltpu.VMEM_SHARED((_M + 1, 16), _f32),
    ],
)(_sc_body)


def _fin_body(part_ref, te_ref, p0_ref, p1_ref, th_ref, mh_ref, bh_ref):
    p = part_ref[0] + part_ref[1]
    t0 = p[:, 0:1]
    t1 = p[:, 1:2]
    te_ref[...] = t0 + t1
    p0_ref[...] = t0
    p1_ref[...] = t0 + t1
    th_ref[...] = p[:, 2:3]
    mh_ref[...] = p[:, 3:4] / p[:, 4:5]
    bh_ref[...] = (jnp.sum(p[:, 3:4], keepdims=True) /
                   jnp.sum(p[:, 4:5], keepdims=True))


def _run_fin(partials):
    m1 = [jax.ShapeDtypeStruct((_M, 1), _f32)] * 5
    return pl.pallas_call(
        _fin_body,
        out_shape=m1 + [jax.ShapeDtypeStruct((1, 1), _f32)],
    )(partials)


def kernel(all_features, mol_index, n_molecules, W0, W1, b1):
    mol = mol_index.astype(jnp.int32)
    mol3 = jnp.concatenate(
        [mol, jnp.full((_NPAD - _N,), _M, jnp.int32)]).reshape(_NW, _GRP, _BATCH)
    zeros = jnp.zeros((_M + 1, 16), _f32)
    dep = (jnp.asarray(n_molecules, jnp.int32) - _M).astype(_f32).reshape(1, 1)
    b1r = b1.astype(_f32).reshape(1, 1)
    atomen, ahier, vals = _run_tc(dep, b1r, all_features,
                                  W0.astype(_f32), W1.astype(_f32))
    partials = _sc_segsum(vals, mol3, zeros)
    te, p0, p1, th, mh, bh = _run_fin(partials)
    return (te, atomen, (p0, p1), th, ahier, mh, jnp.reshape(bh, ()))


# speedup vs baseline: 1.0595x; 1.0595x over previous
import functools

import jax
import jax.numpy as jnp
from jax.experimental import pallas as pl
from jax.experimental.pallas import tpu as pltpu
from jax.experimental.pallas import tpu_sc as plsc

_N = 160000
_D = 128
_M = 1024
_B = 8000
_NB = _N // _B

_NW = 32
_GRP = 40
_BATCH = 128
_CHUNK = _GRP * _BATCH
_NPAD = _NW * _CHUNK
_f32 = jnp.float32


def _tc_body(dep_ref, b1_ref, feats_ref, w0_ref, w1_ref,
             atomen_ref, ahier_ref, vals_ref):
    f0 = feats_ref[0]
    f1 = feats_ref[1]
    w0 = w0_ref[...]
    w1 = w1_ref[...]
    dep = dep_ref[0, 0]
    b1 = b1_ref[0, 0]
    f0b = f0.astype(jnp.bfloat16).astype(_f32)
    f1b = f1.astype(jnp.bfloat16).astype(_f32)
    w0b = w0.astype(jnp.bfloat16).astype(_f32)
    w1b = w1.astype(jnp.bfloat16).astype(_f32)
    pe0 = jnp.sum(f0b * w0b, axis=1, keepdims=True) + dep
    pe1 = jnp.sum(f1b * w1b, axis=1, keepdims=True) + b1
    e0s = pe0 * pe0
    e1s = pe1 * pe1
    den = e0s + e1s
    hier = e1s / den
    atomen_ref[...] = pe0 + pe1
    ahier_ref[...] = hier
    vals_ref[...] = jnp.concatenate(
        [pe0, pe1, hier, e1s, den, jnp.zeros((_B, 11), _f32)], axis=1)


def _run_tc(dep, b1r, feats, W0, W1):
    return pl.pallas_call(
        _tc_body,
        grid=(_NB,),
        in_specs=[
            pl.BlockSpec(memory_space=pltpu.SMEM),
            pl.BlockSpec(memory_space=pltpu.SMEM),
            pl.BlockSpec((2, _B, _D), lambda i: (0, i, 0)),
            pl.BlockSpec((1, _D), lambda i: (0, 0)),
            pl.BlockSpec((1, _D), lambda i: (0, 0)),
        ],
        out_specs=[
            pl.BlockSpec((_B, 1), lambda i: (i, 0)),
            pl.BlockSpec((_B, 1), lambda i: (i, 0)),
            pl.BlockSpec((_B, 16), lambda i: (i, 0)),
        ],
        out_shape=[
            jax.ShapeDtypeStruct((_N, 1), _f32),
            jax.ShapeDtypeStruct((_N, 1), _f32),
            jax.ShapeDtypeStruct((_NPAD, 16), _f32),
        ],
    )(dep, b1r, feats, W0, W1)


def _sc_body(vals_hbm, mol_hbm, zeros_hbm, out_hbm, idx_v, vals_v, acc_sh):
    c = jax.lax.axis_index("c")
    s = jax.lax.axis_index("s")
    wid = s * 2 + c
    pltpu.sync_copy(mol_hbm.at[wid], idx_v)
    pltpu.sync_copy(vals_hbm.at[pl.ds(wid * _CHUNK, _CHUNK)], vals_v)

    @pl.when(s == 0)
    def _zero():
        pltpu.sync_copy(zeros_hbm, acc_sh)

    plsc.subcore_barrier()

    def _grp(g, carry):
        pltpu.sync_copy(vals_v.at[pl.ds(g * _BATCH, _BATCH)],
                        acc_sh.at[idx_v.at[g]], add=True)
        return carry

    jax.lax.fori_loop(0, _GRP, _grp, 0)
    plsc.subcore_barrier()

    @pl.when(s == 0)
    def _flush():
        pltpu.sync_copy(acc_sh.at[pl.ds(0, _M)], out_hbm.at[c])


_sc_segsum = functools.partial(
    pl.kernel,
    out_type=jax.ShapeDtypeStruct((2, _M, 16), _f32),
    mesh=plsc.VectorSubcoreMesh(core_axis_name="c", subcore_axis_name="s"),
    compiler_params=pltpu.CompilerParams(use_tc_tiling_on_sc=False),
    scratch_types=[
        pltpu.VMEM((_GRP, _BATCH), jnp.int32),
        pltpu.VMEM((_CHUNK, 16), _f32),
        pltpu.VMEM_SHARED((_M + 1, 16), _f32),
    ],
)(_sc_body)


def _fin_body(part_ref, te_ref, p0_ref, p1_ref, th_ref, mh_ref, bh_ref):
    p = part_ref[0] + part_ref[1]
    t0 = p[:, 0:1]
    t1 = p[:, 1:2]
    te_ref[...] = t0 + t1
    p0_ref[...] = t0
    p1_ref[...] = t0 + t1
    th_ref[...] = p[:, 2:3]
    mh_ref[...] = p[:, 3:4] / p[:, 4:5]
    bh_ref[...] = (jnp.sum(p[:, 3:4], keepdims=True) /
                   jnp.sum(p[:, 4:5], keepdims=True))


def _run_fin(partials):
    m1 = [jax.ShapeDtypeStruct((_M, 1), _f32)] * 5
    return pl.pallas_call(
        _fin_body,
        out_shape=m1 + [jax.ShapeDtypeStruct((1, 1), _f32)],
    )(partials)


def kernel(all_features, mol_index, n_molecules, W0, W1, b1):
    mol = mol_index.astype(jnp.int32)
    mol3 = jnp.concatenate(
        [mol, jnp.full((_NPAD - _N,), _M, jnp.int32)]).reshape(_NW, _GRP, _BATCH)
    zeros = jnp.zeros((_M + 1, 16), _f32)
    dep = (jnp.asarray(n_molecules, jnp.int32) - _M).astype(_f32).reshape(1, 1)
    b1r = b1.astype(_f32).reshape(1, 1)
    atomen, ahier, vals = _run_tc(dep, b1r, all_features,
                                  W0.astype(_f32), W1.astype(_f32))
    partials = _sc_segsum(vals, mol3, zeros)
    te, p0, p1, th, mh, bh = _run_fin(partials)
    return (te, atomen, (p0, p1), th, ahier, mh, jnp.reshape(bh, ()))
